# Initial kernel scaffold; baseline (speedup 1.0000x reference)
import jax
import jax.numpy as jnp
from jax.experimental import pallas as pl


def kernel(feat, coord, W, b):
    raise NotImplementedError("write your pallas kernel here")


# speedup vs baseline: 1.8901x; 1.8901x over previous
import jax
import jax.numpy as jnp
from jax.experimental import pallas as pl
from jax.experimental.pallas import tpu as pltpu

_N = 100000
_C = 512
_NCLS = 20
_NSP = 8
_SP0 = 8
_K = 25
_BN = 2000
_NB = _N // _BN


def _labels_from_logits(logits):
    m = jnp.max(logits, axis=1, keepdims=True)
    lane = jax.lax.broadcasted_iota(jnp.int32, logits.shape, 1)
    return jnp.min(jnp.where(logits == m, lane, _NCLS), axis=1, keepdims=True)


def _logits_kernel(feat_ref, coord_ref, w_ref, b_ref, logits_ref, part_ref):
    logits = jnp.dot(feat_ref[...], w_ref[...],
                     preferred_element_type=jnp.float32) + b_ref[...]
    logits_ref[...] = logits
    label = _labels_from_logits(logits)
    cls = jax.lax.broadcasted_iota(jnp.int32, (_BN, _NSP), 1) + _SP0
    onehot = (label == cls).astype(jnp.float32)
    coord = coord_ref[...]
    sx = jnp.sum(coord[:, 0:1] * onehot, axis=0, keepdims=True)
    sy = jnp.sum(coord[:, 1:2] * onehot, axis=0, keepdims=True)
    sz = jnp.sum(coord[:, 2:3] * onehot, axis=0, keepdims=True)
    cnt = jnp.sum(onehot, axis=0, keepdims=True)
    part_ref[0] = jnp.concatenate([sx, sy, sz, cnt], axis=1)


def _centroids(parts):
    s = jnp.sum(parts, axis=0, keepdims=True)
    cnt = jnp.maximum(s[:, 3 * _NSP:4 * _NSP], 1.0)
    return s[:, 0:_NSP] / cnt, s[:, _NSP:2 * _NSP] / cnt, s[:, 2 * _NSP:3 * _NSP] / cnt


def _dist_kernel(coord_ref, part_ref, d2_ref, candv_ref, candc_ref):
    cx, cy, cz = _centroids(part_ref[...].reshape(_NB, 4 * _NSP))
    coord = coord_ref[...]
    dx = coord[:, 0:1] - cx
    dy = coord[:, 1:2] - cy
    dz = coord[:, 2:3] - cz
    d2 = dx * dx + dy * dy + dz * dz
    d2_ref[...] = d2
    d2m = d2
    for i in range(_K):
        vmin = jnp.min(d2m, axis=0, keepdims=True)
        eq = d2m == vmin
        candv_ref[0, 0:1, _NSP * i:_NSP * (i + 1)] = vmin
        candc_ref[0, 0:1, _NSP * i:_NSP * (i + 1)] = jnp.sum(
            eq.astype(jnp.float32), axis=0, keepdims=True)
        d2m = jnp.where(eq, jnp.inf, d2m)


def _cumsum_rows(x):
    n = x.shape[0]
    s = 1
    while s < n:
        shifted = jnp.concatenate(
            [jnp.zeros((s, x.shape[1]), x.dtype), x[:-s, :]], axis=0)
        x = x + shifted
        s *= 2
    return x


def _apply_kernel(logits_ref, d2_ref, candv_ref, candc_ref, out_ref, scr_ref):
    pid = pl.program_id(0)

    @pl.when(pid == 0)
    def _merge():
        candv = candv_ref[...].reshape(_NB, _K * _NSP)
        candc = candc_ref[...].reshape(_NB, _K * _NSP)
        cv = jnp.concatenate(
            [candv[:, _NSP * i:_NSP * (i + 1)] for i in range(_K)], axis=0)
        cc = jnp.concatenate(
            [candc[:, _NSP * i:_NSP * (i + 1)] for i in range(_K)], axis=0)
        active = jnp.ones((1, _NSP), dtype=jnp.bool_)
        cum = jnp.zeros((1, _NSP), dtype=jnp.float32)
        thr = jnp.zeros((1, _NSP), dtype=jnp.float32)
        rr = jnp.zeros((1, _NSP), dtype=jnp.float32)
        for _ in range(_K):
            vmin = jnp.min(cv, axis=0, keepdims=True)
            eqm = cv == vmin
            csum = jnp.sum(jnp.where(eqm, cc, 0.0), axis=0, keepdims=True)
            newcum = cum + csum
            hit = active & (newcum >= _K)
            thr = jnp.where(hit, vmin, thr)
            rr = jnp.where(hit, _K - cum, rr)
            active = active & (~hit)
            cum = newcum
            cv = jnp.where(eqm, jnp.inf, cv)
        scr_ref[0:1, :] = thr
        scr_ref[1:2, :] = rr
        scr_ref[2:3, :] = jnp.zeros((1, _NSP), jnp.float32)

    thr = scr_ref[0:1, :]
    rr = scr_ref[1:2, :]
    ties = scr_ref[2:3, :]
    d2 = d2_ref[...]
    lt = d2 < thr
    eqf = (d2 == thr).astype(jnp.float32)
    cume = _cumsum_rows(eqf)
    sel_tie = (eqf > 0.0) & ((ties + cume) <= rr)
    nearest = lt | sel_tie
    scr_ref[2:3, :] = ties + jnp.sum(eqf, axis=0, keepdims=True)

    logits = logits_ref[...]
    label = _labels_from_logits(logits)
    cls = jax.lax.broadcasted_iota(jnp.int32, (_BN, _NSP), 1) + _SP0
    onehot = label == cls
    near_lbl = jnp.sum(jnp.where(onehot & nearest, 1.0, 0.0),
                       axis=1, keepdims=True)
    inspare = (label >= _SP0) & (label < _SP0 + _NSP)
    reset = inspare & (near_lbl == 0.0)
    lane = jax.lax.broadcasted_iota(jnp.int32, logits.shape, 1)
    target = jnp.where(lane == 1, 10.0, 0.0).astype(jnp.float32)
    out_ref[...] = jnp.where(reset, target, logits)


def kernel(feat, coord, W, b):
    b2 = b.reshape(1, _NCLS)
    logits, parts = pl.pallas_call(
        _logits_kernel,
        grid=(_NB,),
        in_specs=[
            pl.BlockSpec((_BN, _C), lambda i: (i, 0)),
            pl.BlockSpec((_BN, 3), lambda i: (i, 0)),
            pl.BlockSpec((_C, _NCLS), lambda i: (0, 0)),
            pl.BlockSpec((1, _NCLS), lambda i: (0, 0)),
        ],
        out_specs=[
            pl.BlockSpec((_BN, _NCLS), lambda i: (i, 0)),
            pl.BlockSpec((1, 1, 4 * _NSP), lambda i: (i, 0, 0)),
        ],
        out_shape=[
            jax.ShapeDtypeStruct((_N, _NCLS), jnp.float32),
            jax.ShapeDtypeStruct((_NB, 1, 4 * _NSP), jnp.float32),
        ],
        compiler_params=pltpu.CompilerParams(
            dimension_semantics=("parallel",)),
    )(feat, coord, W, b2)

    d2, candv, candc = pl.pallas_call(
        _dist_kernel,
        grid=(_NB,),
        in_specs=[
            pl.BlockSpec((_BN, 3), lambda i: (i, 0)),
            pl.BlockSpec((_NB, 1, 4 * _NSP), lambda i: (0, 0, 0)),
        ],
        out_specs=[
            pl.BlockSpec((_BN, _NSP), lambda i: (i, 0)),
            pl.BlockSpec((1, 1, _K * _NSP), lambda i: (i, 0, 0)),
            pl.BlockSpec((1, 1, _K * _NSP), lambda i: (i, 0, 0)),
        ],
        out_shape=[
            jax.ShapeDtypeStruct((_N, _NSP), jnp.float32),
            jax.ShapeDtypeStruct((_NB, 1, _K * _NSP), jnp.float32),
            jax.ShapeDtypeStruct((_NB, 1, _K * _NSP), jnp.float32),
        ],
        compiler_params=pltpu.CompilerParams(
            dimension_semantics=("parallel",)),
    )(coord, parts)

    out = pl.pallas_call(
        _apply_kernel,
        grid=(_NB,),
        in_specs=[
            pl.BlockSpec((_BN, _NCLS), lambda i: (i, 0)),
            pl.BlockSpec((_BN, _NSP), lambda i: (i, 0)),
            pl.BlockSpec((_NB, 1, _K * _NSP), lambda i: (0, 0, 0)),
            pl.BlockSpec((_NB, 1, _K * _NSP), lambda i: (0, 0, 0)),
        ],
        out_specs=pl.BlockSpec((_BN, _NCLS), lambda i: (i, 0)),
        out_shape=jax.ShapeDtypeStruct((_N, _NCLS), jnp.float32),
        scratch_shapes=[pltpu.VMEM((8, _NSP), jnp.float32)],
        compiler_params=pltpu.CompilerParams(
            dimension_semantics=("arbitrary",)),
    )(logits, d2, candv, candc)
    return out


# speedup vs baseline: 3.8038x; 2.0125x over previous
import jax
import jax.numpy as jnp
from jax.experimental import pallas as pl
from jax.experimental.pallas import tpu as pltpu

_N = 100000
_C = 512
_NCLS = 20
_NSP = 8
_SP0 = 8
_K = 25
_BN = 2000
_NB = _N // _BN


def _logits_kernel(feat_ref, coord_ref, w_ref, b_ref,
                   logits_ref, labt_ref, part_ref):
    logits = jnp.dot(feat_ref[...], w_ref[...],
                     preferred_element_type=jnp.float32) + b_ref[...]
    logits_ref[...] = logits
    m = jnp.max(logits, axis=1, keepdims=True)
    lane = jax.lax.broadcasted_iota(jnp.int32, logits.shape, 1)
    label = jnp.min(jnp.where(logits == m, lane, _NCLS),
                    axis=1, keepdims=True)
    labt_ref[0] = jnp.transpose(label.astype(jnp.float32))
    cls = jax.lax.broadcasted_iota(jnp.int32, (_BN, _NSP), 1) + _SP0
    onehot = (label == cls).astype(jnp.float32)
    coordaug = jnp.concatenate(
        [coord_ref[...], jnp.ones((_BN, 1), jnp.float32)], axis=1)
    part_ref[0] = jax.lax.dot_general(
        coordaug, onehot, (((0,), (0,)), ((), ())),
        preferred_element_type=jnp.float32)


def _cumsum_lanes(x):
    n = x.shape[1]
    s = 1
    while s < n:
        shifted = jnp.concatenate(
            [jnp.zeros((x.shape[0], s), x.dtype), x[:, :-s]], axis=1)
        x = x + shifted
        s *= 2
    return x


def _postproc_kernel(coordt_ref, part_ref, logits_ref, labt_ref, out_ref,
                     d2_scr, candv_scr, candc_scr, scr_ref):
    p = pl.program_id(0)
    i = pl.program_id(1)

    @pl.when((p == 0) & (i == 0))
    def _cents():
        s = part_ref[0]
        for j in range(1, _NB):
            s = s + part_ref[j]
        t = jnp.transpose(s)
        cnt = jnp.maximum(t[:, 3:4], 1.0)
        scr_ref[:, 3:6] = t[:, 0:3] / cnt

    @pl.when(p == 0)
    def _dist():
        cx = scr_ref[:, 3:4]
        cy = scr_ref[:, 4:5]
        cz = scr_ref[:, 5:6]
        dx = coordt_ref[0, 0:1, :] - cx
        dy = coordt_ref[0, 1:2, :] - cy
        dz = coordt_ref[0, 2:3, :] - cz
        d2 = dx * dx + dy * dy + dz * dz
        d2_scr[i] = d2
        d2m = d2
        for it in range(_K):
            vmin = jnp.min(d2m, axis=1, keepdims=True)
            eq = d2m == vmin
            candv_scr[pl.ds(_NSP * i, _NSP), it:it + 1] = vmin
            candc_scr[pl.ds(_NSP * i, _NSP), it:it + 1] = jnp.sum(
                eq.astype(jnp.float32), axis=1, keepdims=True)
            d2m = jnp.where(eq, jnp.inf, d2m)

    @pl.when((p == 1) & (i == 0))
    def _merge():
        cv = jnp.concatenate(
            [candv_scr[_NSP * j:_NSP * (j + 1), :] for j in range(_NB)],
            axis=1)
        cc = jnp.concatenate(
            [candc_scr[_NSP * j:_NSP * (j + 1), :] for j in range(_NB)],
            axis=1)
        active = jnp.ones((_NSP, 1), dtype=jnp.bool_)
        cum = jnp.zeros((_NSP, 1), dtype=jnp.float32)
        thr = jnp.zeros((_NSP, 1), dtype=jnp.float32)
        rr = jnp.zeros((_NSP, 1), dtype=jnp.float32)
        for _ in range(_K):
            vmin = jnp.min(cv, axis=1, keepdims=True)
            eqm = cv == vmin
            csum = jnp.sum(jnp.where(eqm, cc, 0.0), axis=1, keepdims=True)
            newcum = cum + csum
            hit = active & (newcum >= _K)
            thr = jnp.where(hit, vmin, thr)
            rr = jnp.where(hit, _K - cum, rr)
            active = active & (~hit)
            cum = newcum
            cv = jnp.where(eqm, jnp.inf, cv)
        scr_ref[:, 0:1] = thr
        scr_ref[:, 1:2] = rr
        scr_ref[:, 2:3] = jnp.zeros((_NSP, 1), jnp.float32)

    @pl.when(p == 1)
    def _apply():
        thr = scr_ref[:, 0:1]
        rr = scr_ref[:, 1:2]
        ties = scr_ref[:, 2:3]
        d2 = d2_scr[i]
        lt = d2 < thr
        eqf = (d2 == thr).astype(jnp.float32)
        cume = _cumsum_lanes(eqf)
        sel_tie = (eqf > 0.0) & ((ties + cume) <= rr)
        nearest = lt | sel_tie
        scr_ref[:, 2:3] = ties + jnp.sum(eqf, axis=1, keepdims=True)

        labt = labt_ref[0]
        cls = jax.lax.broadcasted_iota(
            jnp.int32, (_NSP, 1), 0).astype(jnp.float32) + float(_SP0)
        onehot = labt == cls
        near_lbl = jnp.sum(jnp.where(onehot & nearest, 1.0, 0.0),
                           axis=0, keepdims=True)
        inspare = (labt >= float(_SP0)) & (labt < float(_SP0 + _NSP))
        resetf = jnp.where(inspare & (near_lbl == 0.0), 1.0, 0.0)
        reset = jnp.transpose(resetf) > 0.0
        logits = logits_ref[...]
        lane = jax.lax.broadcasted_iota(jnp.int32, logits.shape, 1)
        target = jnp.where(lane == 1, 10.0, 0.0).astype(jnp.float32)
        out_ref[...] = jnp.where(reset, target, logits)


def kernel(feat, coord, W, b):
    b2 = b.reshape(1, _NCLS)
    coordt = coord.reshape(_NB, _BN, 3).transpose(0, 2, 1)
    logits, labt, parts = pl.pallas_call(
        _logits_kernel,
        grid=(_NB,),
        in_specs=[
            pl.BlockSpec((_BN, _C), lambda i: (i, 0)),
            pl.BlockSpec((_BN, 3), lambda i: (i, 0)),
            pl.BlockSpec((_C, _NCLS), lambda i: (0, 0)),
            pl.BlockSpec((1, _NCLS), lambda i: (0, 0)),
        ],
        out_specs=[
            pl.BlockSpec((_BN, _NCLS), lambda i: (i, 0)),
            pl.BlockSpec((1, 1, _BN), lambda i: (i, 0, 0)),
            pl.BlockSpec((1, 4, _NSP), lambda i: (i, 0, 0)),
        ],
        out_shape=[
            jax.ShapeDtypeStruct((_N, _NCLS), jnp.float32),
            jax.ShapeDtypeStruct((_NB, 1, _BN), jnp.float32),
            jax.ShapeDtypeStruct((_NB, 4, _NSP), jnp.float32),
        ],
        compiler_params=pltpu.CompilerParams(
            dimension_semantics=("parallel",)),
    )(feat, coord, W, b2)

    out = pl.pallas_call(
        _postproc_kernel,
        grid=(2, _NB),
        in_specs=[
            pl.BlockSpec((1, 3, _BN), lambda p, i: (i * (1 - p), 0, 0)),
            pl.BlockSpec((_NB, 4, _NSP), lambda p, i: (0, 0, 0)),
            pl.BlockSpec((_BN, _NCLS), lambda p, i: (i * p, 0)),
            pl.BlockSpec((1, 1, _BN), lambda p, i: (i * p, 0, 0)),
        ],
        out_specs=pl.BlockSpec((_BN, _NCLS), lambda p, i: (i * p, 0)),
        out_shape=jax.ShapeDtypeStruct((_N, _NCLS), jnp.float32),
        scratch_shapes=[
            pltpu.VMEM((_NB, _NSP, _BN), jnp.float32),
            pltpu.VMEM((_NB * _NSP, _K), jnp.float32),
            pltpu.VMEM((_NB * _NSP, _K), jnp.float32),
            pltpu.VMEM((_NSP, 8), jnp.float32),
        ],
        compiler_params=pltpu.CompilerParams(
            dimension_semantics=("arbitrary", "arbitrary")),
    )(coordt, parts, logits, labt)
    return out


# speedup vs baseline: 4.7093x; 1.2380x over previous
import jax
import jax.numpy as jnp
from jax.experimental import pallas as pl
from jax.experimental.pallas import tpu as pltpu

_N = 100000
_C = 512
_NCLS = 20
_NSP = 8
_SP0 = 8
_K = 25
_BN = 2000
_NB = _N // _BN
_BP = 4000
_NP = _N // _BP


def _logits_kernel(feat_ref, coord_ref, w_ref, b_ref,
                   logits_ref, labt_ref, part_ref):
    logits = jnp.dot(feat_ref[...], w_ref[...],
                     preferred_element_type=jnp.float32) + b_ref[...]
    logits_ref[...] = logits
    m = jnp.max(logits, axis=1, keepdims=True)
    lane = jax.lax.broadcasted_iota(jnp.int32, logits.shape, 1)
    label = jnp.min(jnp.where(logits == m, lane, _NCLS),
                    axis=1, keepdims=True)
    labt_ref[0] = jnp.transpose(label.astype(jnp.float32))
    cls = jax.lax.broadcasted_iota(jnp.int32, (_BN, _NSP), 1) + _SP0
    onehot = (label == cls).astype(jnp.float32)
    coordaug = jnp.concatenate(
        [coord_ref[...], jnp.ones((_BN, 1), jnp.float32)], axis=1)
    part_ref[0] = jax.lax.dot_general(
        coordaug, onehot, (((0,), (0,)), ((), ())),
        preferred_element_type=jnp.float32)


def _cumsum_lanes(x):
    n = x.shape[1]
    s = 1
    while s < n:
        shifted = jnp.concatenate(
            [jnp.zeros((x.shape[0], s), x.dtype), x[:, :-s]], axis=1)
        x = x + shifted
        s *= 2
    return x


def _postproc_kernel(coordt_ref, part_ref, logits_ref, labt_ref, out_ref,
                     d2_scr, candv_scr, candc_scr, scr_ref):
    p = pl.program_id(0)
    i = pl.program_id(1)

    @pl.when((p == 0) & (i == 0))
    def _cents():
        s = part_ref[0]
        for j in range(1, _NB):
            s = s + part_ref[j]
        t = jnp.transpose(s)
        cnt = jnp.maximum(t[:, 3:4], 1.0)
        scr_ref[:, 3:6] = t[:, 0:3] / cnt

    @pl.when(p == 0)
    def _dist():
        cx = scr_ref[:, 3:4]
        cy = scr_ref[:, 4:5]
        cz = scr_ref[:, 5:6]
        dx = coordt_ref[0, 0:1, :] - cx
        dy = coordt_ref[0, 1:2, :] - cy
        dz = coordt_ref[0, 2:3, :] - cz
        d2 = dx * dx + dy * dy + dz * dz
        d2_scr[i] = d2
        d2m = d2
        for it in range(_K):
            vmin = jnp.min(d2m, axis=1, keepdims=True)
            eq = d2m == vmin
            candv_scr[pl.ds(_NSP * i, _NSP), it:it + 1] = vmin
            candc_scr[pl.ds(_NSP * i, _NSP), it:it + 1] = jnp.sum(
                eq.astype(jnp.float32), axis=1, keepdims=True)
            d2m = jnp.where(eq, jnp.inf, d2m)

    @pl.when((p == 1) & (i == 0))
    def _merge():
        cv = jnp.concatenate(
            [candv_scr[_NSP * j:_NSP * (j + 1), :] for j in range(_NP)],
            axis=1)
        cc = jnp.concatenate(
            [candc_scr[_NSP * j:_NSP * (j + 1), :] for j in range(_NP)],
            axis=1)
        active = jnp.ones((_NSP, 1), dtype=jnp.bool_)
        cum = jnp.zeros((_NSP, 1), dtype=jnp.float32)
        thr = jnp.zeros((_NSP, 1), dtype=jnp.float32)
        rr = jnp.zeros((_NSP, 1), dtype=jnp.float32)
        for _ in range(_K):
            vmin = jnp.min(cv, axis=1, keepdims=True)
            eqm = cv == vmin
            csum = jnp.sum(jnp.where(eqm, cc, 0.0), axis=1, keepdims=True)
            newcum = cum + csum
            hit = active & (newcum >= _K)
            thr = jnp.where(hit, vmin, thr)
            rr = jnp.where(hit, _K - cum, rr)
            active = active & (~hit)
            cum = newcum
            cv = jnp.where(eqm, jnp.inf, cv)
        scr_ref[:, 0:1] = thr
        scr_ref[:, 1:2] = rr
        scr_ref[:, 2:3] = jnp.zeros((_NSP, 1), jnp.float32)

    @pl.when(p == 1)
    def _apply():
        thr = scr_ref[:, 0:1]
        rr = scr_ref[:, 1:2]
        ties = scr_ref[:, 2:3]
        d2 = d2_scr[i]
        lt = d2 < thr
        eqf = (d2 == thr).astype(jnp.float32)
        neq = jnp.sum(eqf, axis=1, keepdims=True)

        def _tie_path():
            cume = _cumsum_lanes(eqf)
            return jnp.where(
                (eqf > 0.0) & ((ties + cume) <= rr), 1.0, 0.0)

        sel_tie = jax.lax.cond(
            jnp.sum(neq) > 0.0, _tie_path, lambda: jnp.zeros_like(eqf))
        nearest = lt | (sel_tie > 0.0)
        scr_ref[:, 2:3] = ties + neq

        labt = labt_ref[0]
        cls = jax.lax.broadcasted_iota(
            jnp.int32, (_NSP, 1), 0).astype(jnp.float32) + float(_SP0)
        onehot = labt == cls
        near_lbl = jnp.sum(jnp.where(onehot & nearest, 1.0, 0.0),
                           axis=0, keepdims=True)
        inspare = (labt >= float(_SP0)) & (labt < float(_SP0 + _NSP))
        resetf = jnp.where(inspare & (near_lbl == 0.0), 1.0, 0.0)
        reset = jnp.transpose(resetf) > 0.0
        logits = logits_ref[...]
        lane = jax.lax.broadcasted_iota(jnp.int32, logits.shape, 1)
        target = jnp.where(lane == 1, 10.0, 0.0).astype(jnp.float32)
        out_ref[...] = jnp.where(reset, target, logits)


def kernel(feat, coord, W, b):
    b2 = b.reshape(1, _NCLS)
    coordt = coord.reshape(_NP, _BP, 3).transpose(0, 2, 1)
    logits, labt, parts = pl.pallas_call(
        _logits_kernel,
        grid=(_NB,),
        in_specs=[
            pl.BlockSpec((_BN, _C), lambda i: (i, 0)),
            pl.BlockSpec((_BN, 3), lambda i: (i, 0)),
            pl.BlockSpec((_C, _NCLS), lambda i: (0, 0)),
            pl.BlockSpec((1, _NCLS), lambda i: (0, 0)),
        ],
        out_specs=[
            pl.BlockSpec((_BN, _NCLS), lambda i: (i, 0)),
            pl.BlockSpec((1, 1, _BN), lambda i: (i, 0, 0)),
            pl.BlockSpec((1, 4, _NSP), lambda i: (i, 0, 0)),
        ],
        out_shape=[
            jax.ShapeDtypeStruct((_N, _NCLS), jnp.float32),
            jax.ShapeDtypeStruct((_NB, 1, _BN), jnp.float32),
            jax.ShapeDtypeStruct((_NB, 4, _NSP), jnp.float32),
        ],
        compiler_params=pltpu.CompilerParams(
            dimension_semantics=("parallel",)),
    )(feat, coord, W, b2)

    labt4 = labt.reshape(_NP, 1, _BP)
    out = pl.pallas_call(
        _postproc_kernel,
        grid=(2, _NP),
        in_specs=[
            pl.BlockSpec((1, 3, _BP), lambda p, i: (i * (1 - p), 0, 0)),
            pl.BlockSpec((_NB, 4, _NSP), lambda p, i: (0, 0, 0)),
            pl.BlockSpec((_BP, _NCLS), lambda p, i: (i * p, 0)),
            pl.BlockSpec((1, 1, _BP), lambda p, i: (i * p, 0, 0)),
        ],
        out_specs=pl.BlockSpec((_BP, _NCLS), lambda p, i: (i * p, 0)),
        out_shape=jax.ShapeDtypeStruct((_N, _NCLS), jnp.float32),
        scratch_shapes=[
            pltpu.VMEM((_NP, _NSP, _BP), jnp.float32),
            pltpu.VMEM((_NP * _NSP, _K), jnp.float32),
            pltpu.VMEM((_NP * _NSP, _K), jnp.float32),
            pltpu.VMEM((_NSP, 8), jnp.float32),
        ],
        compiler_params=pltpu.CompilerParams(
            dimension_semantics=("arbitrary", "arbitrary")),
    )(coordt, parts, logits, labt4)
    return out
